# Initial kernel scaffold; baseline (speedup 1.0000x reference)
import jax
import jax.numpy as jnp
from jax.experimental import pallas as pl


def kernel(inputs, embedding):
    raise NotImplementedError("write your pallas kernel here")


# speedup vs baseline: 1.2266x; 1.2266x over previous
import functools

import jax
import jax.numpy as jnp
from jax import lax
from jax.experimental import pallas as pl
from jax.experimental.pallas import tpu as pltpu
from jax.experimental.pallas import tpu_sc as plsc

_NUM_CODES = 8192
_DIM = 32
_COMMITMENT = 0.25

_TN = 256


def _argmin_body(x_ref, et_ref, idx_ref, loss_ref):
    i = pl.program_id(0)
    x = x_ref[...]
    et = et_ref[...]
    xq = (2.0 * x).astype(jnp.bfloat16).astype(jnp.float32)
    s2 = jnp.dot(xq, et)
    x2 = jnp.sum(x * x, axis=1, keepdims=True)
    e2 = jnp.sum(et * et, axis=0, keepdims=True)
    dists = (x2 + e2) - s2
    minv = jnp.min(dists, axis=1, keepdims=True)
    iota = lax.broadcasted_iota(jnp.int32, dists.shape, 1)
    cand = jnp.where(dists == minv, iota, _NUM_CODES)
    idx_ref[...] = jnp.min(cand, axis=1)

    @pl.when(i == 0)
    def _():
        loss_ref[...] = jnp.zeros((1, 1), jnp.float32)

    loss_ref[...] += jnp.sum(minv).reshape(1, 1)


def _tc_argmin(flat, et):
    n = flat.shape[0]
    grid = (n // _TN,)
    return pl.pallas_call(
        _argmin_body,
        grid=grid,
        in_specs=[
            pl.BlockSpec((_TN, _DIM), lambda i: (i, 0)),
            pl.BlockSpec((_DIM, _NUM_CODES), lambda i: (0, 0)),
        ],
        out_specs=[
            pl.BlockSpec((_TN,), lambda i: (i,)),
            pl.BlockSpec((1, 1), lambda i: (0, 0)),
        ],
        out_shape=[
            jax.ShapeDtypeStruct((n,), jnp.int32),
            jax.ShapeDtypeStruct((1, 1), jnp.float32),
        ],
    )(flat, et)


def _make_sc_gather(n_rows):
    info = plsc.get_sparse_core_info()
    nw = info.num_cores * info.num_subcores
    b_per_w = n_rows // nw
    chunk = 128
    n_chunks = b_per_w // chunk
    mesh = plsc.VectorSubcoreMesh(core_axis_name="c", subcore_axis_name="s")

    @functools.partial(
        pl.kernel,
        mesh=mesh,
        compiler_params=pltpu.CompilerParams(use_tc_tiling_on_sc=False),
        out_type=jax.ShapeDtypeStruct((n_rows, _DIM), jnp.float32),
        scratch_types=[
            pltpu.VMEM((n_chunks, chunk), jnp.int32),
            pltpu.VMEM((b_per_w, _DIM), jnp.float32),
            pltpu.SemaphoreType.DMA,
        ],
    )
    def gather(table_hbm, idx_hbm, out_hbm, idx_v, rows_v, sem):
        wid = lax.axis_index("s") * info.num_cores + lax.axis_index("c")
        base = wid * b_per_w
        pltpu.sync_copy(idx_hbm.at[wid], idx_v)
        copies = []
        for j in range(n_chunks):
            copies.append(pltpu.async_copy(
                table_hbm.at[idx_v.at[j]],
                rows_v.at[pl.ds(j * chunk, chunk), :],
                sem))
        for c in copies:
            c.wait()
        pltpu.sync_copy(rows_v, out_hbm.at[pl.ds(base, b_per_w)])

    def run(embedding, indices):
        idx3 = indices.reshape(nw, n_chunks, chunk)
        return gather(embedding, idx3)

    return run


def kernel(inputs, embedding):
    b, t, d = inputs.shape
    n = b * t
    flat = inputs.reshape(n, d)
    et = embedding.T
    indices, loss_sum = _tc_argmin(flat, et)
    quantized = _make_sc_gather(n)(embedding, indices).reshape(inputs.shape)
    loss = _COMMITMENT * (loss_sum[0, 0] / jnp.float32(n * d))
    quantized_st = inputs + (quantized - inputs)
    encoding_indices = indices.reshape(b, t)
    return quantized_st, loss, encoding_indices


# speedup vs baseline: 1.2742x; 1.0388x over previous
import functools

import jax
import jax.numpy as jnp
from jax import lax
from jax.experimental import pallas as pl
from jax.experimental.pallas import tpu as pltpu
from jax.experimental.pallas import tpu_sc as plsc

_NUM_CODES = 8192
_DIM = 32
_COMMITMENT = 0.25

_TN = 512


def _argmin_body(x_ref, et_ref, idx_ref, loss_ref):
    i = pl.program_id(0)
    x = x_ref[...]
    et = et_ref[...]
    xq = (2.0 * x).astype(jnp.bfloat16).astype(jnp.float32)
    s2 = jnp.dot(xq, et)
    x2 = jnp.sum(x * x, axis=1, keepdims=True)
    e2 = jnp.sum(et * et, axis=0, keepdims=True)
    dists = (x2 + e2) - s2
    minv = jnp.min(dists, axis=1, keepdims=True)
    iota = lax.broadcasted_iota(jnp.int32, dists.shape, 1)
    cand = jnp.where(dists == minv, iota, _NUM_CODES)
    idx_ref[...] = jnp.min(cand, axis=1)

    @pl.when(i == 0)
    def _():
        loss_ref[...] = jnp.zeros((1, 1), jnp.float32)

    loss_ref[...] += jnp.sum(minv).reshape(1, 1)


def _tc_argmin(flat, et):
    n = flat.shape[0]
    grid = (n // _TN,)
    return pl.pallas_call(
        _argmin_body,
        grid=grid,
        in_specs=[
            pl.BlockSpec((_TN, _DIM), lambda i: (i, 0)),
            pl.BlockSpec((_DIM, _NUM_CODES), lambda i: (0, 0)),
        ],
        out_specs=[
            pl.BlockSpec((_TN,), lambda i: (i,)),
            pl.BlockSpec((1, 1), lambda i: (0, 0)),
        ],
        out_shape=[
            jax.ShapeDtypeStruct((n,), jnp.int32),
            jax.ShapeDtypeStruct((1, 1), jnp.float32),
        ],
    )(flat, et)


def _make_sc_gather(n_rows):
    info = plsc.get_sparse_core_info()
    nw = info.num_cores * info.num_subcores
    b_per_w = n_rows // nw
    chunk = 128
    n_chunks = b_per_w // chunk
    mesh = plsc.VectorSubcoreMesh(core_axis_name="c", subcore_axis_name="s")

    @functools.partial(
        pl.kernel,
        mesh=mesh,
        compiler_params=pltpu.CompilerParams(use_tc_tiling_on_sc=False),
        out_type=jax.ShapeDtypeStruct((n_rows, _DIM), jnp.float32),
        scratch_types=[
            pltpu.VMEM((n_chunks, chunk), jnp.int32),
            pltpu.VMEM((b_per_w, _DIM), jnp.float32),
            pltpu.SemaphoreType.DMA,
        ],
    )
    def gather(table_hbm, idx_hbm, out_hbm, idx_v, rows_v, sem):
        wid = lax.axis_index("s") * info.num_cores + lax.axis_index("c")
        base = wid * b_per_w
        pltpu.sync_copy(idx_hbm.at[wid], idx_v)
        copies = []
        for j in range(n_chunks):
            copies.append(pltpu.async_copy(
                table_hbm.at[idx_v.at[j]],
                rows_v.at[pl.ds(j * chunk, chunk), :],
                sem))
        for c in copies:
            c.wait()
        pltpu.sync_copy(rows_v, out_hbm.at[pl.ds(base, b_per_w)])

    def run(embedding, indices):
        idx3 = indices.reshape(nw, n_chunks, chunk)
        return gather(embedding, idx3)

    return run


def kernel(inputs, embedding):
    b, t, d = inputs.shape
    n = b * t
    flat = inputs.reshape(n, d)
    et = embedding.T
    indices, loss_sum = _tc_argmin(flat, et)
    quantized = _make_sc_gather(n)(embedding, indices).reshape(inputs.shape)
    loss = _COMMITMENT * (loss_sum[0, 0] / jnp.float32(n * d))
    quantized_st = inputs + (quantized - inputs)
    encoding_indices = indices.reshape(b, t)
    return quantized_st, loss, encoding_indices
